# baseline (speedup 1.0000x reference)
import functools

import jax
import jax.numpy as jnp
from jax import lax
from jax.experimental import pallas as pl
from jax.experimental.pallas import tpu as pltpu
from jax.experimental.pallas import tpu_sc as plsc

_NC = 2
_NS = 16
_NW = _NC * _NS
_LANES = 16

_B, _L, _D = 4, 2048, 1024
_LPW = _L // _NW
_CH = 32
_NCH = _LPW // _CH
_VECS = _D // _LANES


def _sc_body(x_hbm, tab_hbm, out_hbm, tbuf, xbuf):
    wid = lax.axis_index("s") * _NC + lax.axis_index("c")
    lbase = wid * _LPW
    for ch in range(_NCH):
        base = lbase + ch * _CH
        pltpu.sync_copy(tab_hbm.at[pl.ds(base, _CH + 8)], tbuf)
        for b in range(_B):
            pltpu.sync_copy(x_hbm.at[b, pl.ds(base, _CH)], xbuf)

            @plsc.parallel_loop(0, _CH * _VECS, unroll=8)
            def _add(i):
                r = i // _VECS
                c = (i % _VECS) * _LANES
                xbuf[r, pl.ds(c, _LANES)] = (
                    xbuf[r, pl.ds(c, _LANES)] + tbuf[r + 1, pl.ds(c, _LANES)]
                )

            pltpu.sync_copy(xbuf, out_hbm.at[b, pl.ds(base, _CH)])


@jax.jit
def _sc_kernel(x, pos_table):
    tab = jnp.pad(pos_table, ((0, _L + 8 - pos_table.shape[0]), (0, 0)))
    mesh = plsc.VectorSubcoreMesh(core_axis_name="c", subcore_axis_name="s")
    return pl.kernel(
        _sc_body,
        out_type=jax.ShapeDtypeStruct((_B, _L, _D), jnp.float32),
        mesh=mesh,
        scratch_types=[
            pltpu.VMEM((_CH + 8, _D), jnp.float32),
            pltpu.VMEM((_CH, _D), jnp.float32),
        ],
    )(x, tab)


def kernel(x, pos_table):
    return _sc_kernel(x, pos_table)


# speedup vs baseline: 1.2132x; 1.2132x over previous
import jax
import jax.numpy as jnp
from jax import lax
from jax.experimental import pallas as pl
from jax.experimental.pallas import tpu as pltpu
from jax.experimental.pallas import tpu_sc as plsc

_NC = 2
_NS = 16
_NW = _NC * _NS
_LANES = 16

_B, _L, _D = 4, 2048, 1024
_LPW = _L // _NW
_R = 8
_NSUB = _LPW // _R
_NIT = _NSUB * _B
_VECS = _D // _LANES


def _sc_body(x_hbm, tab_hbm, out_hbm,
             tbuf, xin0, xin1, ob0, ob1, sin0, sin1, sout0, sout1):
    wid = lax.axis_index("s") * _NC + lax.axis_index("c")
    lbase = wid * _LPW
    pltpu.sync_copy(tab_hbm.at[pl.ds(lbase, _LPW + 8)], tbuf)

    xin, ob = [xin0, xin1], [ob0, ob1]
    sin, sout = [sin0, sin1], [sout0, sout1]

    def x_src(it):
        sub, b = divmod(it, _B)
        return x_hbm.at[b, pl.ds(lbase + sub * _R, _R)]

    def o_dst(it):
        sub, b = divmod(it, _B)
        return out_hbm.at[b, pl.ds(lbase + sub * _R, _R)]

    in_d, out_d = {}, {}
    for it in range(2):
        in_d[it] = pltpu.async_copy(x_src(it), xin[it % 2], sin[it % 2])
    for it in range(_NIT):
        s = it % 2
        in_d[it].wait()
        if it >= 2:
            out_d[it - 2].wait()
        sub = it // _B
        trow = sub * _R + 1
        xb, obuf = xin[s], ob[s]

        @plsc.parallel_loop(0, _R * _VECS, unroll=8)
        def _add(i, trow=trow, xb=xb, obuf=obuf):
            r = i // _VECS
            c = (i % _VECS) * _LANES
            obuf[r, pl.ds(c, _LANES)] = (
                xb[r, pl.ds(c, _LANES)] + tbuf[trow + r, pl.ds(c, _LANES)]
            )

        out_d[it] = pltpu.async_copy(obuf, o_dst(it), sout[s])
        if it + 2 < _NIT:
            in_d[it + 2] = pltpu.async_copy(x_src(it + 2), xb, sin[s])
    out_d[_NIT - 2].wait()
    out_d[_NIT - 1].wait()


@jax.jit
def _sc_kernel(x, pos_table):
    tab = jnp.pad(pos_table, ((0, _L + 8 - pos_table.shape[0]), (0, 0)))
    mesh = plsc.VectorSubcoreMesh(core_axis_name="c", subcore_axis_name="s")
    return pl.kernel(
        _sc_body,
        out_type=jax.ShapeDtypeStruct((_B, _L, _D), jnp.float32),
        mesh=mesh,
        scratch_types=[
            pltpu.VMEM((_LPW + 8, _D), jnp.float32),
            pltpu.VMEM((_R, _D), jnp.float32),
            pltpu.VMEM((_R, _D), jnp.float32),
            pltpu.VMEM((_R, _D), jnp.float32),
            pltpu.VMEM((_R, _D), jnp.float32),
            pltpu.SemaphoreType.DMA,
            pltpu.SemaphoreType.DMA,
            pltpu.SemaphoreType.DMA,
            pltpu.SemaphoreType.DMA,
        ],
    )(x, tab)


def kernel(x, pos_table):
    return _sc_kernel(x, pos_table)


# speedup vs baseline: 1.2915x; 1.0646x over previous
import jax
import jax.numpy as jnp
from jax import lax
from jax.experimental import pallas as pl
from jax.experimental.pallas import tpu as pltpu
from jax.experimental.pallas import tpu_sc as plsc

_NC = 2
_NS = 16
_NW = _NC * _NS
_LANES = 16

_B, _L, _D = 4, 2048, 1024
_LPW = _L // _NW
_R = 8
_NSUB = _LPW // _R
_NIT = _NSUB * _B
_VECS = _D // _LANES


def _sc_body(x_hbm, tab_hbm, out_hbm,
             tbuf, xin0, xin1, ob0, ob1, sin0, sin1, sout0, sout1):
    wid = lax.axis_index("s") * _NC + lax.axis_index("c")
    lbase = wid * _LPW
    pltpu.sync_copy(tab_hbm.at[pl.ds(lbase, _LPW)], tbuf.at[pl.ds(0, _LPW)])
    pltpu.sync_copy(tab_hbm.at[pl.ds(lbase + _LPW, 1)], tbuf.at[pl.ds(_LPW, 1)])

    xin, ob = [xin0, xin1], [ob0, ob1]
    sin, sout = [sin0, sin1], [sout0, sout1]

    def x_src(it):
        sub, b = divmod(it, _B)
        return x_hbm.at[b, pl.ds(lbase + sub * _R, _R)]

    def o_dst(it):
        sub, b = divmod(it, _B)
        return out_hbm.at[b, pl.ds(lbase + sub * _R, _R)]

    in_d, out_d = {}, {}
    for it in range(2):
        in_d[it] = pltpu.async_copy(x_src(it), xin[it % 2], sin[it % 2])
    for it in range(_NIT):
        s = it % 2
        in_d[it].wait()
        if it >= 2:
            out_d[it - 2].wait()
        sub = it // _B
        trow = sub * _R + 1
        xb, obuf = xin[s], ob[s]

        @plsc.parallel_loop(0, _R * _VECS, unroll=8)
        def _add(i, trow=trow, xb=xb, obuf=obuf):
            r = i // _VECS
            c = (i % _VECS) * _LANES
            obuf[r, pl.ds(c, _LANES)] = (
                xb[r, pl.ds(c, _LANES)] + tbuf[trow + r, pl.ds(c, _LANES)]
            )

        out_d[it] = pltpu.async_copy(obuf, o_dst(it), sout[s])
        if it + 2 < _NIT:
            in_d[it + 2] = pltpu.async_copy(x_src(it + 2), xb, sin[s])
    out_d[_NIT - 2].wait()
    out_d[_NIT - 1].wait()


@jax.jit
def _sc_kernel(x, pos_table):
    mesh = plsc.VectorSubcoreMesh(core_axis_name="c", subcore_axis_name="s")
    return pl.kernel(
        _sc_body,
        out_type=jax.ShapeDtypeStruct((_B, _L, _D), jnp.float32),
        mesh=mesh,
        scratch_types=[
            pltpu.VMEM((_LPW + 1, _D), jnp.float32),
            pltpu.VMEM((_R, _D), jnp.float32),
            pltpu.VMEM((_R, _D), jnp.float32),
            pltpu.VMEM((_R, _D), jnp.float32),
            pltpu.VMEM((_R, _D), jnp.float32),
            pltpu.SemaphoreType.DMA,
            pltpu.SemaphoreType.DMA,
            pltpu.SemaphoreType.DMA,
            pltpu.SemaphoreType.DMA,
        ],
    )(x, pos_table)


def kernel(x, pos_table):
    return _sc_kernel(x, pos_table)


# speedup vs baseline: 2.3807x; 1.8433x over previous
import jax
import jax.numpy as jnp
from jax.experimental import pallas as pl


_BL = 512


def _body(x_ref, tab_ref, o_ref):
    j = pl.program_id(0)
    win = tab_ref[pl.ds(j * _BL, _BL + 8), :]
    pe = win[1:_BL + 1]
    o_ref[...] = x_ref[...] + pe[None]


def kernel(x, pos_table):
    B, L, D = x.shape
    Tp = L + 8
    tab = jnp.pad(pos_table, ((0, Tp - pos_table.shape[0]), (0, 0)))
    grid = (L // _BL,)
    return pl.pallas_call(
        _body,
        grid=grid,
        in_specs=[
            pl.BlockSpec((B, _BL, D), lambda j: (0, j, 0)),
            pl.BlockSpec((Tp, D), lambda j: (0, 0)),
        ],
        out_specs=pl.BlockSpec((B, _BL, D), lambda j: (0, j, 0)),
        out_shape=jax.ShapeDtypeStruct(x.shape, x.dtype),
    )(x, tab)


# speedup vs baseline: 2.7677x; 1.1626x over previous
import jax
import jax.numpy as jnp
from jax.experimental import pallas as pl


_BL = 512


def _body(x_ref, tab_ref, o_ref):
    j = pl.program_id(1)
    win = tab_ref[pl.ds(j * _BL, _BL), :]
    nxt = tab_ref[pl.ds(pl.multiple_of((j + 1) * _BL, 8), 1), :]
    pe = jnp.concatenate([win[1:], nxt], axis=0)
    o_ref[...] = x_ref[...] + pe[None]


def kernel(x, pos_table):
    B, L, D = x.shape
    T = pos_table.shape[0]
    grid = (B, L // _BL)
    return pl.pallas_call(
        _body,
        grid=grid,
        in_specs=[
            pl.BlockSpec((1, _BL, D), lambda b, j: (b, j, 0)),
            pl.BlockSpec((T, D), lambda b, j: (0, 0)),
        ],
        out_specs=pl.BlockSpec((1, _BL, D), lambda b, j: (b, j, 0)),
        out_shape=jax.ShapeDtypeStruct(x.shape, x.dtype),
    )(x, pos_table)


# speedup vs baseline: 3.0299x; 1.0947x over previous
import jax
import jax.numpy as jnp
from jax.experimental import pallas as pl


_RB = 1024


def _body(x_ref, tab_ref, o_ref):
    g = pl.program_id(0)
    nblk_per_seq = 2048 // _RB
    j = g % nblk_per_seq
    win = tab_ref[pl.ds(j * _RB, _RB), :]
    nxt = tab_ref[pl.ds(pl.multiple_of((j + 1) * _RB, 8), 1), :]
    pe = jnp.concatenate([win[1:], nxt], axis=0)
    o_ref[...] = x_ref[...] + pe


def kernel(x, pos_table):
    B, L, D = x.shape
    T = pos_table.shape[0]
    xf = x.reshape(B * L, D)
    grid = (B * L // _RB,)
    out = pl.pallas_call(
        _body,
        grid=grid,
        in_specs=[
            pl.BlockSpec((_RB, D), lambda g: (g, 0)),
            pl.BlockSpec((T, D), lambda g: (0, 0)),
        ],
        out_specs=pl.BlockSpec((_RB, D), lambda g: (g, 0)),
        out_shape=jax.ShapeDtypeStruct((B * L, D), x.dtype),
    )(xf, pos_table)
    return out.reshape(B, L, D)


# speedup vs baseline: 3.2293x; 1.0658x over previous
import jax
import jax.numpy as jnp
from jax.experimental import pallas as pl
from jax.experimental.pallas import tpu as pltpu

_B, _L, _D = 4, 2048, 1024
_T = _L + 1
_C = 1024
_NCK = _B * _L // _C
_NIB = 3
_NOB = 2


def _body(x_hbm, tab_hbm, o_hbm, tabv, xbufs, obufs, tsem, isems, osems):
    tcopy = pltpu.make_async_copy(tab_hbm, tabv, tsem)
    tcopy.start()

    def in_copy(k):
        return pltpu.make_async_copy(
            x_hbm.at[pl.ds(k * _C, _C)], xbufs.at[k % _NIB], isems.at[k % _NIB])

    def out_copy(k):
        return pltpu.make_async_copy(
            obufs.at[k % _NOB], o_hbm.at[pl.ds(k * _C, _C)], osems.at[k % _NOB])

    for k in range(_NIB):
        in_copy(k).start()
    tcopy.wait()

    for k in range(_NCK):
        si, so = k % _NIB, k % _NOB
        in_copy(k).wait()
        if k >= _NOB:
            out_copy(k - _NOB).wait()
        joff = (k % (_L // _C)) * _C
        win = tabv[pl.ds(joff, _C), :]
        nxt = tabv[pl.ds(joff + _C, 1), :]
        pe = jnp.concatenate([win[1:], nxt], axis=0)
        obufs[so] = xbufs[si] + pe
        out_copy(k).start()
        if k + _NIB < _NCK:
            in_copy(k + _NIB).start()
    for k in range(_NCK - _NOB, _NCK):
        out_copy(k).wait()


def kernel(x, pos_table):
    B, L, D = x.shape
    xf = x.reshape(B * L, D)
    out = pl.pallas_call(
        _body,
        in_specs=[
            pl.BlockSpec(memory_space=pltpu.MemorySpace.HBM),
            pl.BlockSpec(memory_space=pltpu.MemorySpace.HBM),
        ],
        out_specs=pl.BlockSpec(memory_space=pltpu.MemorySpace.HBM),
        out_shape=jax.ShapeDtypeStruct((B * L, D), x.dtype),
        scratch_shapes=[
            pltpu.VMEM((_T, _D), jnp.float32),
            pltpu.VMEM((_NIB, _C, _D), jnp.float32),
            pltpu.VMEM((_NOB, _C, _D), jnp.float32),
            pltpu.SemaphoreType.DMA,
            pltpu.SemaphoreType.DMA((_NIB,)),
            pltpu.SemaphoreType.DMA((_NOB,)),
        ],
    )(xf, pos_table)
    return out.reshape(B, L, D)


# speedup vs baseline: 3.4347x; 1.0636x over previous
import jax
import jax.numpy as jnp
from jax.experimental import pallas as pl
from jax.experimental.pallas import tpu as pltpu

_B, _L, _D = 4, 2048, 1024
_T = _L + 1
_C = 2048
_NCK = _B * _L // _C
_NIB = 3
_NOB = 2


def _body(x_hbm, tab_hbm, o_hbm, tabv, xbufs, obufs, tsem, isems, osems):
    tcopy = pltpu.make_async_copy(tab_hbm, tabv, tsem)
    tcopy.start()

    def in_copy(k):
        return pltpu.make_async_copy(
            x_hbm.at[pl.ds(k * _C, _C)], xbufs.at[k % _NIB], isems.at[k % _NIB])

    def out_copy(k):
        return pltpu.make_async_copy(
            obufs.at[k % _NOB], o_hbm.at[pl.ds(k * _C, _C)], osems.at[k % _NOB])

    for k in range(_NIB):
        in_copy(k).start()
    tcopy.wait()

    for k in range(_NCK):
        si, so = k % _NIB, k % _NOB
        in_copy(k).wait()
        if k >= _NOB:
            out_copy(k - _NOB).wait()
        joff = (k % (_L // _C)) * _C
        win = tabv[pl.ds(joff, _C), :]
        nxt = tabv[pl.ds(joff + _C, 1), :]
        pe = jnp.concatenate([win[1:], nxt], axis=0)
        obufs[so] = xbufs[si] + pe
        out_copy(k).start()
        if k + _NIB < _NCK:
            in_copy(k + _NIB).start()
    for k in range(_NCK - _NOB, _NCK):
        out_copy(k).wait()


def kernel(x, pos_table):
    B, L, D = x.shape
    xf = x.reshape(B * L, D)
    out = pl.pallas_call(
        _body,
        in_specs=[
            pl.BlockSpec(memory_space=pltpu.MemorySpace.HBM),
            pl.BlockSpec(memory_space=pltpu.MemorySpace.HBM),
        ],
        out_specs=pl.BlockSpec(memory_space=pltpu.MemorySpace.HBM),
        out_shape=jax.ShapeDtypeStruct((B * L, D), x.dtype),
        scratch_shapes=[
            pltpu.VMEM((_T, _D), jnp.float32),
            pltpu.VMEM((_NIB, _C, _D), jnp.float32),
            pltpu.VMEM((_NOB, _C, _D), jnp.float32),
            pltpu.SemaphoreType.DMA,
            pltpu.SemaphoreType.DMA((_NIB,)),
            pltpu.SemaphoreType.DMA((_NOB,)),
        ],
    )(xf, pos_table)
    return out.reshape(B, L, D)
